# baseline (speedup 1.0000x reference)
import jax
import jax.numpy as jnp
from jax import lax
from jax.experimental import pallas as pl
from jax.experimental.pallas import tpu as pltpu
from jax.experimental.pallas import tpu_sc as plsc

N = 10000
DEG = 32
IN_FEATS = 128
OUT_FEATS = 128

BN = 400
K = 1600
NW = 32
CHUNK = 8
NCHUNKS_TOT = K // CHUNK
NLG = IN_FEATS // 16


def _sc_mean_body(mail_hbm, h_hbm, inbuf, hbuf):
    c = lax.axis_index("c")
    s = lax.axis_index("s")
    wid = s * 2 + c
    nchunks = (NCHUNKS_TOT - wid + NW - 1) // NW

    def chunk_body(t, carry):
        nodebase = (wid + t * NW) * CHUNK
        pltpu.sync_copy(mail_hbm.at[pl.ds(nodebase * DEG, CHUNK * DEG)], inbuf)
        for n in range(CHUNK):
            accs = [inbuf[n * DEG, pl.ds(l * 16, 16)] for l in range(NLG)]
            for d in range(1, DEG):
                for l in range(NLG):
                    accs[l] = accs[l] + inbuf[n * DEG + d, pl.ds(l * 16, 16)]
            for l in range(NLG):
                hbuf[n, pl.ds(l * 16, 16)] = accs[l] * (1.0 / DEG)
        pltpu.sync_copy(hbuf, h_hbm.at[pl.ds(nodebase, CHUNK)])
        return carry

    lax.fori_loop(0, nchunks, chunk_body, 0)


_sc_mean = pl.kernel(
    _sc_mean_body,
    out_type=jax.ShapeDtypeStruct((K, IN_FEATS), jnp.float32),
    mesh=plsc.VectorSubcoreMesh(core_axis_name="c", subcore_axis_name="s"),
    scratch_types=[
        pltpu.VMEM((CHUNK * DEG, IN_FEATS), jnp.float32),
        pltpu.VMEM((CHUNK, IN_FEATS), jnp.float32),
    ],
)


def _tc_body(x_ref, w_ref, b_ref, o_ref):
    x = x_ref[...]
    h = jnp.sum(x, axis=1) * (1.0 / DEG)
    o = lax.dot_general(h, w_ref[...], (((1,), (1,)), ((), ())),
                        preferred_element_type=jnp.float32)
    o_ref[...] = o + b_ref[...]


def _fc_body(h_ref, w_ref, b_ref, full_ref, o_ref):
    del full_ref
    o = lax.dot_general(h_ref[...], w_ref[...], (((1,), (1,)), ((), ())),
                        preferred_element_type=jnp.float32)
    o_ref[...] = o + b_ref[...]


@jax.jit
def kernel(mailbox_h, W, b):
    b2 = b.reshape(1, OUT_FEATS)

    h_sc = _sc_mean(mailbox_h.reshape(N * DEG, IN_FEATS))

    out_main = pl.pallas_call(
        _tc_body,
        grid=((N - K) // BN,),
        in_specs=[
            pl.BlockSpec((BN, DEG, IN_FEATS), lambda i: (i + K // BN, 0, 0)),
            pl.BlockSpec((OUT_FEATS, IN_FEATS), lambda i: (0, 0)),
            pl.BlockSpec((1, OUT_FEATS), lambda i: (0, 0)),
        ],
        out_specs=pl.BlockSpec((BN, OUT_FEATS), lambda i: (i + K // BN, 0)),
        out_shape=jax.ShapeDtypeStruct((N, OUT_FEATS), jnp.float32),
    )(mailbox_h, W, b2)

    out = pl.pallas_call(
        _fc_body,
        grid=(K // BN,),
        in_specs=[
            pl.BlockSpec((BN, IN_FEATS), lambda i: (i, 0)),
            pl.BlockSpec((OUT_FEATS, IN_FEATS), lambda i: (0, 0)),
            pl.BlockSpec((1, OUT_FEATS), lambda i: (0, 0)),
            pl.BlockSpec(memory_space=pl.ANY),
        ],
        out_specs=pl.BlockSpec((BN, OUT_FEATS), lambda i: (i, 0)),
        out_shape=jax.ShapeDtypeStruct((N, OUT_FEATS), jnp.float32),
        input_output_aliases={3: 0},
    )(h_sc, W, b2, out_main)
    return out


# speedup vs baseline: 1.6993x; 1.6993x over previous
import jax
import jax.numpy as jnp
from jax import lax
from jax.experimental import pallas as pl

N = 10000
DEG = 32
IN_FEATS = 128
OUT_FEATS = 128

BN = 400


def _body(x_ref, w_ref, b_ref, o_ref):
    x = x_ref[...]
    h = jnp.sum(x, axis=1)
    o = lax.dot_general(h, w_ref[...], (((1,), (1,)), ((), ())),
                        preferred_element_type=jnp.float32)
    o_ref[...] = o + b_ref[...]


@jax.jit
def kernel(mailbox_h, W, b):
    b2 = b.reshape(1, OUT_FEATS)
    Ws = W * (1.0 / DEG)
    out = pl.pallas_call(
        _body,
        grid=(N // BN,),
        in_specs=[
            pl.BlockSpec((BN, DEG, IN_FEATS), lambda i: (i, 0, 0)),
            pl.BlockSpec((OUT_FEATS, IN_FEATS), lambda i: (0, 0)),
            pl.BlockSpec((1, OUT_FEATS), lambda i: (0, 0)),
        ],
        out_specs=pl.BlockSpec((BN, OUT_FEATS), lambda i: (i, 0)),
        out_shape=jax.ShapeDtypeStruct((N, OUT_FEATS), jnp.float32),
    )(mailbox_h, Ws, b2)
    return out


# speedup vs baseline: 1.7430x; 1.0257x over previous
import functools

import jax
import jax.numpy as jnp
from jax import lax
from jax.experimental import pallas as pl

N = 10000
DEG = 32
IN_FEATS = 128
OUT_FEATS = 128

BN = 400


def _body(x_ref, w_ref, b_ref, o_ref):
    x = x_ref[...]
    h = jnp.sum(x, axis=1) * (1.0 / DEG)
    o = lax.dot_general(h, w_ref[...], (((1,), (1,)), ((), ())),
                        preferred_element_type=jnp.float32)
    o_ref[...] = o + b_ref[...]


@functools.partial(jax.jit, static_argnames=())
def kernel(mailbox_h, W, b):
    b2 = b.reshape(1, OUT_FEATS)
    grid = (N // BN,)
    out = pl.pallas_call(
        _body,
        grid=grid,
        in_specs=[
            pl.BlockSpec((BN, DEG, IN_FEATS), lambda i: (i, 0, 0)),
            pl.BlockSpec((OUT_FEATS, IN_FEATS), lambda i: (0, 0)),
            pl.BlockSpec((1, OUT_FEATS), lambda i: (0, 0)),
        ],
        out_specs=pl.BlockSpec((BN, OUT_FEATS), lambda i: (i, 0)),
        out_shape=jax.ShapeDtypeStruct((N, OUT_FEATS), jnp.float32),
    )(mailbox_h, W, b2)
    return out
